# Initial kernel scaffold; baseline (speedup 1.0000x reference)
import jax
import jax.numpy as jnp
from jax.experimental import pallas as pl


def kernel(x, W, b, R, bl):
    raise NotImplementedError("write your pallas kernel here")


# speedup vs baseline: 1.4529x; 1.4529x over previous
import functools

import jax
import jax.numpy as jnp
import numpy as np
from jax import lax
from jax.experimental import pallas as pl
from jax.experimental.pallas import tpu as pltpu
from jax.experimental.pallas import tpu_sc as plsc

_B = 4096
_H = 1024
_D = 256
_NHASH = 16
_NBUCKETS = 1024
_WBUCKET = 4.0
_LANES = 128

_PRIMES = np.array(
    [3, 7, 11, 13, 17, 19, 23, 29, 31, 37, 41, 43, 47, 53, 59, 61],
    dtype=np.int32,
)


_ROWS_A = 512
_GRID_A = _B // _ROWS_A


def _stage_a(x_ref, w_ref, bc_ref, r_ref, blc_ref, pc_ref, outT_ref, bk_ref, h_ref):
    zT = lax.dot_general(
        w_ref[...], x_ref[...], (((0,), (1,)), ((), ())),
        preferred_element_type=jnp.float32,
    )
    outT = jnp.tanh(zT + bc_ref[...])
    outT_ref[...] = outT
    projT = (
        lax.dot_general(
            r_ref[...], outT, (((0,), (0,)), ((), ())),
            preferred_element_type=jnp.float32,
        )
        + blc_ref[...]
    )
    codesT = jnp.floor(projT / _WBUCKET)
    codesT = codesT.astype(jnp.int32).astype(jnp.float32)
    s_col = lax.dot_general(
        codesT, pc_ref[...], (((0,), (1,)), ((), ())),
        preferred_element_type=jnp.float32,
    )
    bk = jnp.mod(s_col.astype(jnp.int32), _NBUCKETS)
    bk_ref[...] = bk
    cols = lax.broadcasted_iota(jnp.int32, (1, _NBUCKETS), 1)
    onehot = jnp.where(bk == cols, 1.0, 0.0)
    h_ref[...] = jnp.sum(onehot, axis=0, keepdims=True).reshape(1, 1, _NBUCKETS)


def _run_stage_a(x, W, bc, Rp, blc, pc):
    return pl.pallas_call(
        _stage_a,
        grid=(_GRID_A,),
        in_specs=[
            pl.BlockSpec((_ROWS_A, _H), lambda i: (i, 0)),
            pl.BlockSpec((_H, _D), lambda i: (0, 0)),
            pl.BlockSpec((_D, 1), lambda i: (0, 0)),
            pl.BlockSpec((_D, _LANES), lambda i: (0, 0)),
            pl.BlockSpec((_LANES, 1), lambda i: (0, 0)),
            pl.BlockSpec((1, _LANES), lambda i: (0, 0)),
        ],
        out_specs=[
            pl.BlockSpec((_D, _ROWS_A), lambda i: (0, i)),
            pl.BlockSpec((_ROWS_A, 1), lambda i: (i, 0)),
            pl.BlockSpec((1, 1, _NBUCKETS), lambda i: (i, 0, 0)),
        ],
        out_shape=[
            jax.ShapeDtypeStruct((_D, _B), jnp.float32),
            jax.ShapeDtypeStruct((_B, 1), jnp.int32),
            jax.ShapeDtypeStruct((_GRID_A, 1, _NBUCKETS), jnp.float32),
        ],
    )(x, W, bc, Rp, blc, pc)


_NC = 2
_NS = 16
_RPH = _B // _NC
_CG = _D // _NS


_TBL = _CG * _NBUCKETS


def _sc_body(outT_hbm, bk_hbm, sums_out, bk_v, rowsT_v, tbl):
    cid = lax.axis_index("c")
    sid = lax.axis_index("s")
    rbase = cid * _RPH
    cbase = sid * _CG

    lanes = jnp.arange(_CG, dtype=jnp.int32)
    lane_rows = lanes * _RPH
    lane_tbl = lanes * _NBUCKETS

    def _zrow(i, carry):
        tbl[pl.ds(i * 16, 16)] = jnp.zeros((16,), jnp.float32)
        return carry

    lax.fori_loop(0, _TBL // 16, _zrow, 0)

    pltpu.sync_copy(bk_hbm.at[pl.ds(rbase, _RPH)], bk_v)
    for c in range(_CG):
        pltpu.sync_copy(
            outT_hbm.at[cbase + c, pl.ds(rbase, _RPH)],
            rowsT_v.at[pl.ds(c * _RPH, _RPH)],
        )

    def _chunk(k, carry):
        bvec = bk_v[pl.ds(k * 16, 16)]
        for j in range(16):
            r = k * 16 + j
            b = bvec[j]
            val = plsc.load_gather(rowsT_v, [lane_rows + r])
            plsc.addupdate_scatter(tbl, [lane_tbl + b], val)
        return carry

    lax.fori_loop(0, _RPH // 16, _chunk, 0)

    pltpu.sync_copy(tbl, sums_out.at[pl.ds(cid * _D * _NBUCKETS + sid * _TBL, _TBL)])


def _run_stage_sc(outT, bucket):
    mesh = plsc.VectorSubcoreMesh(core_axis_name="c", subcore_axis_name="s")
    f = functools.partial(
        pl.kernel,
        out_type=jax.ShapeDtypeStruct((_NC * _D * _NBUCKETS,), jnp.float32),
        mesh=mesh,
        scratch_types=[
            pltpu.VMEM((_RPH,), jnp.int32),
            pltpu.VMEM((_CG * _RPH,), jnp.float32),
            pltpu.VMEM((_TBL,), jnp.float32),
        ],
        compiler_params=pltpu.CompilerParams(needs_layout_passes=False),
    )(_sc_body)
    return f(outT, bucket).reshape(_NC, _D, _NBUCKETS)


_ROWS_B = 512
_GRID_B = _B // _ROWS_B


def _stage_b(outT_ref, sums_ref, hist_ref, q_ref):
    sp = sums_ref[...]
    st = sp[0] + sp[1]
    cnt_row = jnp.sum(hist_ref[...][:, 0, :], axis=0, keepdims=True)
    recip = 1.0 / jnp.maximum(cnt_row, 1.0)
    oT = outT_ref[...]
    dots = lax.dot_general(
        oT, st, (((0,), (0,)), ((), ())), preferred_element_type=jnp.float32
    )
    m2row = lax.dot_general(
        jnp.full((1, _D), 1.0, jnp.float32),
        st * st,
        (((1,), (0,)), ((), ())),
        preferred_element_type=jnp.float32,
    )
    onorm = lax.dot_general(
        oT * oT,
        jnp.full((1, _D), 1.0, jnp.float32),
        (((0,), (1,)), ((), ())),
        preferred_element_type=jnp.float32,
    )
    d2 = jnp.maximum(onorm + m2row * recip * recip - 2.0 * dots * recip, 0.0)
    sim = jnp.where(cnt_row > 0.0, 1.0 / (1.0 + d2), 0.0)
    q_ref[...] = sim / jnp.sum(sim, axis=1, keepdims=True)


def _run_stage_b(outT, sums_p, hist):
    return pl.pallas_call(
        _stage_b,
        grid=(_GRID_B,),
        in_specs=[
            pl.BlockSpec((_D, _ROWS_B), lambda i: (0, i)),
            pl.BlockSpec((_NC, _D, _NBUCKETS), lambda i: (0, 0, 0)),
            pl.BlockSpec((_GRID_A, 1, _NBUCKETS), lambda i: (0, 0, 0)),
        ],
        out_specs=pl.BlockSpec((_ROWS_B, _NBUCKETS), lambda i: (i, 0)),
        out_shape=jax.ShapeDtypeStruct((_B, _NBUCKETS), jnp.float32),
    )(outT, sums_p, hist)


def kernel(x, W, b, R, bl):
    pad = _LANES - _NHASH
    Rp = jnp.pad(R, ((0, 0), (0, pad)))
    blc = jnp.pad(bl, ((0, pad),)).reshape(_LANES, 1)
    pc = jnp.pad(jnp.asarray(_PRIMES), ((0, pad),)).astype(jnp.float32).reshape(1, _LANES)
    bc = b.reshape(_D, 1)
    outT, bucket, hist = _run_stage_a(x, W, bc, Rp, blc, pc)
    sums_p = _run_stage_sc(outT, bucket.reshape(_B))
    return _run_stage_b(outT, sums_p, hist)


# speedup vs baseline: 1.8430x; 1.2685x over previous
import functools

import jax
import jax.numpy as jnp
import numpy as np
from jax import lax
from jax.experimental import pallas as pl
from jax.experimental.pallas import tpu as pltpu
from jax.experimental.pallas import tpu_sc as plsc

_B = 4096
_H = 1024
_D = 256
_NHASH = 16
_NBUCKETS = 1024
_WBUCKET = 4.0
_LANES = 128

_PRIMES = np.array(
    [3, 7, 11, 13, 17, 19, 23, 29, 31, 37, 41, 43, 47, 53, 59, 61],
    dtype=np.int32,
)


_ROWS_A = 512
_GRID_A = _B // _ROWS_A


def _stage_a(x_ref, w_ref, bc_ref, r_ref, blc_ref, pc_ref, outT_ref, bk_ref, h_ref):
    zT = lax.dot_general(
        w_ref[...], x_ref[...], (((0,), (1,)), ((), ())),
        preferred_element_type=jnp.float32,
    )
    outT = jnp.tanh(zT + bc_ref[...])
    outT_ref[...] = outT
    projT = (
        lax.dot_general(
            r_ref[...], outT, (((0,), (0,)), ((), ())),
            preferred_element_type=jnp.float32,
        )
        + blc_ref[...]
    )
    codesT = jnp.floor(projT / _WBUCKET)
    codesT = codesT.astype(jnp.int32).astype(jnp.float32)
    s_col = lax.dot_general(
        codesT, pc_ref[...], (((0,), (1,)), ((), ())),
        preferred_element_type=jnp.float32,
    )
    bk = jnp.mod(s_col.astype(jnp.int32), _NBUCKETS)
    bk_ref[...] = bk
    cols = lax.broadcasted_iota(jnp.int32, (1, _NBUCKETS), 1)
    onehot = jnp.where(bk == cols, 1.0, 0.0)
    h_ref[...] = jnp.sum(onehot, axis=0, keepdims=True).reshape(1, 1, _NBUCKETS)


def _run_stage_a(x, W, bc, Rp, blc, pc):
    return pl.pallas_call(
        _stage_a,
        grid=(_GRID_A,),
        in_specs=[
            pl.BlockSpec((_ROWS_A, _H), lambda i: (i, 0)),
            pl.BlockSpec((_H, _D), lambda i: (0, 0)),
            pl.BlockSpec((_D, 1), lambda i: (0, 0)),
            pl.BlockSpec((_D, _LANES), lambda i: (0, 0)),
            pl.BlockSpec((_LANES, 1), lambda i: (0, 0)),
            pl.BlockSpec((1, _LANES), lambda i: (0, 0)),
        ],
        out_specs=[
            pl.BlockSpec((_D, _ROWS_A), lambda i: (0, i)),
            pl.BlockSpec((_ROWS_A, 1), lambda i: (i, 0)),
            pl.BlockSpec((1, 1, _NBUCKETS), lambda i: (i, 0, 0)),
        ],
        out_shape=[
            jax.ShapeDtypeStruct((_D, _B), jnp.float32),
            jax.ShapeDtypeStruct((_B, 1), jnp.int32),
            jax.ShapeDtypeStruct((_GRID_A, 1, _NBUCKETS), jnp.float32),
        ],
    )(x, W, bc, Rp, blc, pc)


_NC = 2
_NS = 16
_RPH = _B // _NC
_CG = _D // _NS


_TBL = _CG * _NBUCKETS


def _sc_body(outT_hbm, bk_hbm, sums_out, bk_v, rowsT_v, tbl, sem_b, sem_r):
    cid = lax.axis_index("c")
    sid = lax.axis_index("s")
    rbase = cid * _RPH
    cbase = sid * _CG

    lanes = jnp.arange(_CG, dtype=jnp.int32)
    lane_rows = lanes * _RPH
    lane_tbl = lanes * _NBUCKETS

    cp_b = pltpu.async_copy(bk_hbm.at[pl.ds(rbase, _RPH)], bk_v, sem_b)
    cp_r = []
    for c in range(_CG):
        cp_r.append(pltpu.async_copy(
            outT_hbm.at[cbase + c, pl.ds(rbase, _RPH)],
            rowsT_v.at[pl.ds(c * _RPH, _RPH)],
            sem_r,
        ))

    zeros16 = jnp.zeros((16,), jnp.float32)

    @plsc.parallel_loop(0, _TBL // 16, unroll=8)
    def _zrow(i):
        tbl[pl.ds(i * 16, 16)] = zeros16

    cp_b.wait()
    for cp in cp_r:
        cp.wait()

    @plsc.parallel_loop(0, _RPH // 16, unroll=2)
    def _chunk(k):
        bvec = bk_v[pl.ds(k * 16, 16)]
        for j in range(16):
            r = k * 16 + j
            b = bvec[j]
            val = plsc.load_gather(rowsT_v, [lane_rows + r])
            plsc.addupdate_scatter(tbl, [lane_tbl + b], val)

    pltpu.sync_copy(tbl, sums_out.at[pl.ds(cid * _D * _NBUCKETS + sid * _TBL, _TBL)])


def _run_stage_sc(outT, bucket):
    mesh = plsc.VectorSubcoreMesh(core_axis_name="c", subcore_axis_name="s")
    f = functools.partial(
        pl.kernel,
        out_type=jax.ShapeDtypeStruct((_NC * _D * _NBUCKETS,), jnp.float32),
        mesh=mesh,
        scratch_types=[
            pltpu.VMEM((_RPH,), jnp.int32),
            pltpu.VMEM((_CG * _RPH,), jnp.float32),
            pltpu.VMEM((_TBL,), jnp.float32),
            pltpu.SemaphoreType.DMA,
            pltpu.SemaphoreType.DMA,
        ],
        compiler_params=pltpu.CompilerParams(needs_layout_passes=False),
    )(_sc_body)
    return f(outT, bucket).reshape(_NC, _D, _NBUCKETS)


_ROWS_B = 512
_GRID_B = _B // _ROWS_B


def _stage_b(outT_ref, sums_ref, hist_ref, q_ref):
    sp = sums_ref[...]
    st = sp[0] + sp[1]
    cnt_row = jnp.sum(hist_ref[...][:, 0, :], axis=0, keepdims=True)
    recip = 1.0 / jnp.maximum(cnt_row, 1.0)
    oT = outT_ref[...]
    dots = lax.dot_general(
        oT, st, (((0,), (0,)), ((), ())), preferred_element_type=jnp.float32
    )
    m2row = lax.dot_general(
        jnp.full((1, _D), 1.0, jnp.float32),
        st * st,
        (((1,), (0,)), ((), ())),
        preferred_element_type=jnp.float32,
    )
    onorm = lax.dot_general(
        oT * oT,
        jnp.full((1, _D), 1.0, jnp.float32),
        (((0,), (1,)), ((), ())),
        preferred_element_type=jnp.float32,
    )
    d2 = jnp.maximum(onorm + m2row * recip * recip - 2.0 * dots * recip, 0.0)
    sim = jnp.where(cnt_row > 0.0, 1.0 / (1.0 + d2), 0.0)
    q_ref[...] = sim / jnp.sum(sim, axis=1, keepdims=True)


def _run_stage_b(outT, sums_p, hist):
    return pl.pallas_call(
        _stage_b,
        grid=(_GRID_B,),
        in_specs=[
            pl.BlockSpec((_D, _ROWS_B), lambda i: (0, i)),
            pl.BlockSpec((_NC, _D, _NBUCKETS), lambda i: (0, 0, 0)),
            pl.BlockSpec((_GRID_A, 1, _NBUCKETS), lambda i: (0, 0, 0)),
        ],
        out_specs=pl.BlockSpec((_ROWS_B, _NBUCKETS), lambda i: (i, 0)),
        out_shape=jax.ShapeDtypeStruct((_B, _NBUCKETS), jnp.float32),
    )(outT, sums_p, hist)


def kernel(x, W, b, R, bl):
    pad = _LANES - _NHASH
    Rp = jnp.pad(R, ((0, 0), (0, pad)))
    blc = jnp.pad(bl, ((0, pad),)).reshape(_LANES, 1)
    pc = jnp.pad(jnp.asarray(_PRIMES), ((0, pad),)).astype(jnp.float32).reshape(1, _LANES)
    bc = b.reshape(_D, 1)
    outT, bucket, hist = _run_stage_a(x, W, bc, Rp, blc, pc)
    sums_p = _run_stage_sc(outT, bucket.reshape(_B))
    return _run_stage_b(outT, sums_p, hist)
